# baseline (speedup 1.0000x reference)
import functools

import jax
import jax.numpy as jnp
from jax.experimental import pallas as pl

N_USER, N_POST, N_TAG = 10000, 50000, 5000
H = 256
HEADS = 4
OUT = 128

BM = 256


def _ceil_to(x, m):
    return (x + m - 1) // m * m


@functools.partial(jax.jit, static_argnames=('relu',))
def _mm(x, W, b, relu=False):
    M, K = x.shape
    N = W.shape[1]
    Mp = _ceil_to(M, BM)
    xp = jnp.pad(x, ((0, Mp - M), (0, 0)))
    b2 = b.reshape(1, N)

    def body(x_ref, w_ref, b_ref, o_ref):
        acc = jnp.dot(x_ref[...], w_ref[...],
                      preferred_element_type=jnp.float32)
        acc = acc + b_ref[...]
        if relu:
            acc = jnp.maximum(acc, 0.0)
        o_ref[...] = acc

    out = pl.pallas_call(
        body,
        grid=(Mp // BM,),
        in_specs=[
            pl.BlockSpec((BM, K), lambda i: (i, 0)),
            pl.BlockSpec((K, N), lambda i: (0, 0)),
            pl.BlockSpec((1, N), lambda i: (0, 0)),
        ],
        out_specs=pl.BlockSpec((BM, N), lambda i: (i, 0)),
        out_shape=jax.ShapeDtypeStruct((Mp, N), jnp.float32),
    )(xp, W, b2)
    return out[:M]


def _seg_sum(x, seg, n):
    return jax.ops.segment_sum(x, seg, num_segments=n)


def _seg_softmax(logits, seg, n):
    m = jax.ops.segment_max(logits, seg, num_segments=n)
    m = jnp.where(jnp.isfinite(m), m, 0.0)
    e = jnp.exp(logits - m[seg])
    d = _seg_sum(e, seg, n)
    return e / (d[seg] + 1e-16)


def _gat_B(Wmat, avec):
    return jnp.einsum('ihk,hk->ih', Wmat.reshape(H, HEADS, H), avec)


def _gat_Wstack(Wmat):
    return Wmat.reshape(H, HEADS, H).transpose(1, 0, 2).reshape(
        HEADS * H, H) / HEADS


def kernel(x_user, x_post, x_tag, params, ei_authors, ei_likes, ei_comments,
           ei_mentions, ei_has_tag, ei_replies, ei_precedes):
    relu = jax.nn.relu
    p = params
    hu = _mm(x_user, p['in_user']['W'], p['in_user']['b'], relu=True)
    hp = _mm(x_post, p['in_post']['W'], p['in_post']['b'], relu=True)
    ht = _mm(x_tag, p['in_tag']['W'], p['in_tag']['b'], relu=True)

    def inv_cnt(ei, n):
        c = _seg_sum(jnp.ones((ei.shape[1],), jnp.float32), ei[1], n)
        return 1.0 / jnp.maximum(c, 1.0)

    ic_auth = inv_cnt(ei_authors, N_POST)
    ic_likes = inv_cnt(ei_likes, N_POST)
    ic_ment = inv_cnt(ei_mentions, N_USER)
    ic_tag = inv_cnt(ei_has_tag, N_TAG)
    deg = _seg_sum(jnp.ones((ei_replies.shape[1],), jnp.float32),
                   ei_replies[1], N_POST)
    dinv = jnp.where(deg > 0, deg ** -0.5, 0.0)

    def sage_agg(x_src, ei, n_dst, icnt):
        return _seg_sum(x_src[ei[0]], ei[1], n_dst) * icnt[:, None]

    def gat_agg(x_src, a_s, a_d, ei, n_dst):
        src, dst = ei[0], ei[1]
        e = jax.nn.leaky_relu(a_s[src] + a_d[dst], 0.2)
        alpha = _seg_softmax(e, dst, n_dst)
        msg = x_src[src][:, None, :] * alpha[:, :, None]
        return _seg_sum(msg, dst, n_dst).reshape(n_dst, HEADS * H)

    for lay in p['layers']:
        la, ll, lc = lay['authors'], lay['likes'], lay['comments']
        lm, lt_ = lay['mentions'], lay['has_tag']
        lr, lp_ = lay['replies'], lay['precedes']
        tp = p['temporal']

        agg_a = sage_agg(hu, ei_authors, N_POST, ic_auth)
        agg_l = sage_agg(hu, ei_likes, N_POST, ic_likes)
        agg_m = sage_agg(hp, ei_mentions, N_USER, ic_ment)
        agg_t = sage_agg(hp, ei_has_tag, N_TAG, ic_tag)
        hp_scaled = hp * dinv[:, None]
        agg_r = _seg_sum(hp_scaled[ei_replies[0]], ei_replies[1],
                         N_POST) * dinv[:, None]
        as_c = hu @ _gat_B(lc['Ws'], lc['as'])
        ad_c = hp @ _gat_B(lc['Wd'], lc['ad'])
        agg_c = gat_agg(hu, as_c, ad_c, ei_comments, N_POST)
        as_p = hp @ _gat_B(lp_['Ws'], lp_['as'])
        ad_p = hp @ _gat_B(lp_['Wd'], lp_['ad'])
        agg_p = gat_agg(hp, as_p, ad_p, ei_precedes, N_POST)

        xpost = jnp.concatenate(
            [agg_a, agg_l, agg_c, agg_r, agg_p, hp], axis=1)
        Wpost = jnp.concatenate(
            [la['Wl'], ll['Wl'], _gat_Wstack(lc['Ws']), lr['W'],
             _gat_Wstack(lp_['Ws']), la['Wr'] + ll['Wr']], axis=0)
        bpost = la['b'] + ll['b'] + lc['b'] + lr['b'] + lp_['b']
        new_post = _mm(xpost, Wpost, bpost)

        new_user = _mm(jnp.concatenate([agg_m, hu], axis=1),
                       jnp.concatenate([lm['Wl'], lm['Wr']], axis=0),
                       lm['b'], relu=True)
        new_tag = _mm(jnp.concatenate([agg_t, ht], axis=1),
                      jnp.concatenate([lt_['Wl'], lt_['Wr']], axis=0),
                      lt_['b'], relu=True)

        src, dst = ei_precedes[0], ei_precedes[1]
        ts = new_post @ (tp['W'] @ tp['as'])
        td = new_post @ (tp['W'] @ tp['ad'])
        e = jax.nn.leaky_relu(ts[src] + td[dst], 0.2)
        alpha = _seg_softmax(e, dst, N_POST)
        agg_tmp = _seg_sum(alpha[:, None] * new_post[src], dst, N_POST)
        new_post = relu(new_post
                        + _mm(agg_tmp, tp['W'], jnp.zeros((H,), jnp.float32)))

        hu, hp, ht = new_user, new_post, new_tag

    su, sp, st = hu.sum(0), hp.sum(0), ht.sum(0)
    n_all = N_USER + N_POST + N_TAG
    pooled = jnp.concatenate([su / N_USER, sp / N_POST, st / N_TAG,
                              (su + sp + st) / n_all])
    g = relu(pooled @ p['proj']['W1'] + p['proj']['b1'])
    return g @ p['proj']['W2'] + p['proj']['b2']
